# baseline (speedup 1.0000x reference)
import functools

import jax
import jax.numpy as jnp
import numpy as np
from jax import lax
from jax.experimental import pallas as pl
from jax.experimental.pallas import tpu as pltpu
from jax.experimental.pallas import tpu_sc as plsc

N = 4096
K = 1024
D = 32
TOPK = 8

NW = 32
RPW = N // NW
L = 16
SEG = K // L
RADIX = 32
NPASS = 7
RQ = 4

_ZVALS = tuple(float(np.float32(1.0) / np.float32(r + 1)) for r in range(TOPK))


def _dist_body(x_ref, ct_ref, d_ref):
    x = x_ref[...]
    ct = ct_ref[...]
    x2 = jnp.sum(x * x, axis=1, keepdims=True)
    c2 = jnp.sum(ct * ct, axis=0, keepdims=True)
    xc = lax.dot_general(x, ct, (((1,), (0,)), ((), ())),
                         precision=lax.Precision.HIGHEST,
                         preferred_element_type=jnp.float32)
    d_ref[...] = jnp.sqrt(jnp.maximum(x2 + c2 - 2.0 * xc, 0.0))


def _xc_body(k_ref, c_ref, xc_ref):
    oh = jnp.where(k_ref[...] == 0, 1.0, 0.0)
    xc_ref[...] = lax.dot_general(oh, c_ref[...], (((1,), (0,)), ((), ())),
                                  precision=lax.Precision.HIGHEST,
                                  preferred_element_type=jnp.float32)


def _sc_body(d_hbm, isort_hbm, k_hbm, z_hbm,
             dbuf, pay_a, pay_b, offs, sincl, sdig, kbuf, zbuf):
    wid = lax.axis_index("s") * 2 + lax.axis_index("c")
    row0 = wid * RPW
    lane = lax.iota(jnp.int32, L)
    seg_base = lane * SEG
    zero16 = jnp.zeros((L,), jnp.int32)
    one16 = jnp.ones((L,), jnp.int32)
    zvals = jnp.zeros((L,), jnp.float32)
    for r in range(TOPK):
        zvals = jnp.where(lane == r, _ZVALS[r], zvals)

    def quad_body(qi, _):
        row = row0 + qi * RQ
        for j in range(RQ):
            pltpu.sync_copy(d_hbm.at[row + j], dbuf.at[pl.ds(j * K, K)])

        def one_pass(shift, src_pay, dst_pay, first):
            def zero_step(b, _):
                offs[pl.ds(b * L, L)] = zero16
                return 0

            lax.fori_loop(0, RQ * RADIX, zero_step, 0)

            def hist_step(s, _):
                idx = seg_base + s
                for j in range(RQ):
                    if first:
                        pv = idx
                    else:
                        pv = plsc.load_gather(src_pay, [idx + j * K])
                    kv = plsc.bitcast(
                        plsc.load_gather(dbuf, [pv + j * K]), jnp.int32)
                    digit = lax.shift_right_logical(kv, shift) & (RADIX - 1)
                    hidx = digit * L + lane + j * (RADIX * L)
                    plsc.addupdate_scatter(offs, [hidx], one16)
                return 0

            lax.fori_loop(0, SEG, hist_step, 0)

            for j in range(RQ):
                def scan_step(b, _, _j=j):
                    v = offs[pl.ds(_j * (RADIX * L) + b * L, L)]
                    cs = plsc.cumsum(v)
                    sincl[pl.ds(b * L, L)] = cs
                    offs[pl.ds(_j * (RADIX * L) + b * L, L)] = cs - v
                    return 0

                lax.fori_loop(0, RADIX, scan_step, 0)
                t0 = plsc.load_gather(sincl, [lane * L + (L - 1)])
                t1 = plsc.load_gather(
                    sincl, [RADIX * L // 2 + lane * L + (L - 1)])
                cs0 = plsc.cumsum(t0)
                carry0 = jnp.sum(t0)
                sdig[pl.ds(j * RADIX, L)] = cs0 - t0
                sdig[pl.ds(j * RADIX + L, L)] = plsc.cumsum(t1) - t1 + carry0

            def perm_step(s, _):
                idx = seg_base + s
                for j in range(RQ):
                    if first:
                        pv = idx
                    else:
                        pv = plsc.load_gather(src_pay, [idx + j * K])
                    kv = plsc.bitcast(
                        plsc.load_gather(dbuf, [pv + j * K]), jnp.int32)
                    digit = lax.shift_right_logical(kv, shift) & (RADIX - 1)
                    hidx = digit * L + lane + j * (RADIX * L)
                    o1 = plsc.load_gather(offs, [hidx])
                    plsc.store_scatter(offs, [hidx], o1 + 1)
                    pos = o1 + plsc.load_gather(sdig, [digit + j * RADIX])
                    plsc.store_scatter(dst_pay, [pos + j * K], pv)
                return 0

            lax.fori_loop(0, SEG, perm_step, 0)

        one_pass(0, None, pay_a, True)
        for p in range(1, NPASS):
            one_pass(5 * p, (pay_a, pay_b)[(p + 1) % 2],
                     (pay_a, pay_b)[p % 2], False)
        fin_pay = pay_a

        def out_step(s, _):
            pos = s * L + lane
            for j in range(RQ):
                pv = fin_pay[pl.ds(j * K + s * L, L)]
                plsc.store_scatter(kbuf, [pv + j * K], pos)
                zv = jnp.where(s == 0, zvals, jnp.zeros((L,), jnp.float32))
                plsc.store_scatter(zbuf, [pv + j * K], zv)
            return 0

        lax.fori_loop(0, SEG, out_step, 0)
        for j in range(RQ):
            pltpu.sync_copy(fin_pay.at[pl.ds(j * K, K)], isort_hbm.at[row + j])
            pltpu.sync_copy(kbuf.at[pl.ds(j * K, K)], k_hbm.at[row + j])
            pltpu.sync_copy(zbuf.at[pl.ds(j * K, K)], z_hbm.at[row + j])
        return 0

    lax.fori_loop(0, RPW // RQ, quad_body, 0)


@jax.jit
def kernel(x, c):
    ct = c.T
    d = pl.pallas_call(
        _dist_body,
        grid=(N // 256,),
        in_specs=[
            pl.BlockSpec((256, D), lambda i: (i, 0)),
            pl.BlockSpec((D, K), lambda i: (0, 0)),
        ],
        out_specs=pl.BlockSpec((256, K), lambda i: (i, 0)),
        out_shape=jax.ShapeDtypeStruct((N, K), jnp.float32),
    )(x, ct)

    sc_fn = functools.partial(
        pl.kernel,
        mesh=plsc.VectorSubcoreMesh(core_axis_name="c", subcore_axis_name="s"),
        out_type=[
            jax.ShapeDtypeStruct((N, K), jnp.int32),
            jax.ShapeDtypeStruct((N, K), jnp.int32),
            jax.ShapeDtypeStruct((N, K), jnp.float32),
        ],
        scratch_types=[
            pltpu.VMEM((RQ * K,), jnp.float32),
            pltpu.VMEM((RQ * K,), jnp.int32),
            pltpu.VMEM((RQ * K,), jnp.int32),
            pltpu.VMEM((RQ * RADIX * L,), jnp.int32),
            pltpu.VMEM((RADIX * L,), jnp.int32),
            pltpu.VMEM((RQ * RADIX,), jnp.int32),
            pltpu.VMEM((RQ * K,), jnp.int32),
            pltpu.VMEM((RQ * K,), jnp.float32),
        ],
        compiler_params=pltpu.CompilerParams(needs_layout_passes=False),
    )(_sc_body)
    isort, k, z = sc_fn(d)

    x_c = pl.pallas_call(
        _xc_body,
        grid=(N // 256,),
        in_specs=[
            pl.BlockSpec((256, K), lambda i: (i, 0)),
            pl.BlockSpec((K, D), lambda i: (0, 0)),
        ],
        out_specs=pl.BlockSpec((256, D), lambda i: (i, 0)),
        out_shape=jax.ShapeDtypeStruct((N, D), jnp.float32),
    )(k, c)
    return (d, isort, k, z, x_c)


# speedup vs baseline: 1.1304x; 1.1304x over previous
import functools

import jax
import jax.numpy as jnp
import numpy as np
from jax import lax
from jax.experimental import pallas as pl
from jax.experimental.pallas import tpu as pltpu
from jax.experimental.pallas import tpu_sc as plsc

N = 4096
K = 1024
D = 32
TOPK = 8

NW = 32
RPW = N // NW
L = 16
SEG = K // L
RADIX = 32
NPASS = 7
RQ = 4

_ZVALS = tuple(float(np.float32(1.0) / np.float32(r + 1)) for r in range(TOPK))


def _dist_body(x_ref, ct_ref, d_ref):
    x = x_ref[...]
    ct = ct_ref[...]
    x2 = jnp.sum(x * x, axis=1, keepdims=True)
    c2 = jnp.sum(ct * ct, axis=0, keepdims=True)
    xc = lax.dot_general(x, ct, (((1,), (0,)), ((), ())),
                         precision=lax.Precision.HIGHEST,
                         preferred_element_type=jnp.float32)
    d_ref[...] = jnp.sqrt(jnp.maximum(x2 + c2 - 2.0 * xc, 0.0))


def _xc_body(k_ref, c_ref, xc_ref):
    oh = jnp.where(k_ref[...] == 0, 1.0, 0.0)
    xc_ref[...] = lax.dot_general(oh, c_ref[...], (((1,), (0,)), ((), ())),
                                  precision=lax.Precision.HIGHEST,
                                  preferred_element_type=jnp.float32)


def _sc_body(d_hbm, isort_hbm, k_hbm, z_hbm, *scratch):
    dbufs = scratch[0:RQ]
    payas = scratch[RQ:2 * RQ]
    paybs = scratch[2 * RQ:3 * RQ]
    offss = scratch[3 * RQ:4 * RQ]
    sincls = scratch[4 * RQ:5 * RQ]
    sdigs = scratch[5 * RQ:6 * RQ]
    kbufs = scratch[6 * RQ:7 * RQ]
    zbufs = scratch[7 * RQ:8 * RQ]

    wid = lax.axis_index("s") * 2 + lax.axis_index("c")
    row0 = wid * RPW
    lane = lax.iota(jnp.int32, L)
    seg_base = lane * SEG
    zero16 = jnp.zeros((L,), jnp.int32)
    one16 = jnp.ones((L,), jnp.int32)
    zvals = jnp.zeros((L,), jnp.float32)
    for r in range(TOPK):
        zvals = jnp.where(lane == r, _ZVALS[r], zvals)

    def quad_body(qi, _):
        row = row0 + qi * RQ
        for j in range(RQ):
            pltpu.sync_copy(d_hbm.at[row + j], dbufs[j])

        def one_pass(shift, srcs, dsts, first):
            def zero_step(b, _):
                for j in range(RQ):
                    offss[j][pl.ds(b * L, L)] = zero16
                return 0

            lax.fori_loop(0, RADIX, zero_step, 0)

            def hist_step(s, _):
                idx = seg_base + s
                for j in range(RQ):
                    if first:
                        pv = idx
                    else:
                        pv = plsc.load_gather(srcs[j], [idx])
                    kv = plsc.bitcast(
                        plsc.load_gather(dbufs[j], [pv]), jnp.int32)
                    digit = lax.shift_right_logical(kv, shift) & (RADIX - 1)
                    hidx = digit * L + lane
                    plsc.addupdate_scatter(offss[j], [hidx], one16)
                return 0

            lax.fori_loop(0, SEG, hist_step, 0)

            def scan_step(b, _):
                for j in range(RQ):
                    v = offss[j][pl.ds(b * L, L)]
                    cs = plsc.cumsum(v)
                    sincls[j][pl.ds(b * L, L)] = cs
                    offss[j][pl.ds(b * L, L)] = cs - v
                return 0

            lax.fori_loop(0, RADIX, scan_step, 0)
            for j in range(RQ):
                t0 = plsc.load_gather(sincls[j], [lane * L + (L - 1)])
                t1 = plsc.load_gather(
                    sincls[j], [RADIX * L // 2 + lane * L + (L - 1)])
                cs0 = plsc.cumsum(t0)
                carry0 = jnp.sum(t0)
                sdigs[j][pl.ds(0, L)] = cs0 - t0
                sdigs[j][pl.ds(L, L)] = plsc.cumsum(t1) - t1 + carry0

            def perm_step(s, _):
                idx = seg_base + s
                for j in range(RQ):
                    if first:
                        pv = idx
                    else:
                        pv = plsc.load_gather(srcs[j], [idx])
                    kv = plsc.bitcast(
                        plsc.load_gather(dbufs[j], [pv]), jnp.int32)
                    digit = lax.shift_right_logical(kv, shift) & (RADIX - 1)
                    hidx = digit * L + lane
                    o1 = plsc.load_gather(offss[j], [hidx])
                    plsc.store_scatter(offss[j], [hidx], o1 + 1)
                    pos = o1 + plsc.load_gather(sdigs[j], [digit])
                    plsc.store_scatter(dsts[j], [pos], pv)
                return 0

            lax.fori_loop(0, SEG, perm_step, 0)

        one_pass(0, None, payas, True)
        for p in range(1, NPASS):
            one_pass(5 * p, (payas, paybs)[(p + 1) % 2],
                     (payas, paybs)[p % 2], False)
        fin = payas

        def out_step(s, _):
            pos = s * L + lane
            zv = jnp.where(s == 0, zvals, jnp.zeros((L,), jnp.float32))
            for j in range(RQ):
                pv = fin[j][pl.ds(s * L, L)]
                plsc.store_scatter(kbufs[j], [pv], pos)
                plsc.store_scatter(zbufs[j], [pv], zv)
            return 0

        lax.fori_loop(0, SEG, out_step, 0)
        for j in range(RQ):
            pltpu.sync_copy(fin[j], isort_hbm.at[row + j])
            pltpu.sync_copy(kbufs[j], k_hbm.at[row + j])
            pltpu.sync_copy(zbufs[j], z_hbm.at[row + j])
        return 0

    lax.fori_loop(0, RPW // RQ, quad_body, 0)


@jax.jit
def kernel(x, c):
    ct = c.T
    d = pl.pallas_call(
        _dist_body,
        grid=(N // 256,),
        in_specs=[
            pl.BlockSpec((256, D), lambda i: (i, 0)),
            pl.BlockSpec((D, K), lambda i: (0, 0)),
        ],
        out_specs=pl.BlockSpec((256, K), lambda i: (i, 0)),
        out_shape=jax.ShapeDtypeStruct((N, K), jnp.float32),
    )(x, ct)

    scratch = (
        [pltpu.VMEM((K,), jnp.float32) for _ in range(RQ)]
        + [pltpu.VMEM((K,), jnp.int32) for _ in range(RQ)]
        + [pltpu.VMEM((K,), jnp.int32) for _ in range(RQ)]
        + [pltpu.VMEM((RADIX * L,), jnp.int32) for _ in range(RQ)]
        + [pltpu.VMEM((RADIX * L,), jnp.int32) for _ in range(RQ)]
        + [pltpu.VMEM((RADIX,), jnp.int32) for _ in range(RQ)]
        + [pltpu.VMEM((K,), jnp.int32) for _ in range(RQ)]
        + [pltpu.VMEM((K,), jnp.float32) for _ in range(RQ)]
    )
    sc_fn = functools.partial(
        pl.kernel,
        mesh=plsc.VectorSubcoreMesh(core_axis_name="c", subcore_axis_name="s"),
        out_type=[
            jax.ShapeDtypeStruct((N, K), jnp.int32),
            jax.ShapeDtypeStruct((N, K), jnp.int32),
            jax.ShapeDtypeStruct((N, K), jnp.float32),
        ],
        scratch_types=scratch,
        compiler_params=pltpu.CompilerParams(needs_layout_passes=False),
    )(_sc_body)
    isort, k, z = sc_fn(d)

    x_c = pl.pallas_call(
        _xc_body,
        grid=(N // 256,),
        in_specs=[
            pl.BlockSpec((256, K), lambda i: (i, 0)),
            pl.BlockSpec((K, D), lambda i: (0, 0)),
        ],
        out_specs=pl.BlockSpec((256, D), lambda i: (i, 0)),
        out_shape=jax.ShapeDtypeStruct((N, D), jnp.float32),
    )(k, c)
    return (d, isort, k, z, x_c)


# speedup vs baseline: 1.1618x; 1.0277x over previous
import functools

import jax
import jax.numpy as jnp
import numpy as np
from jax import lax
from jax.experimental import pallas as pl
from jax.experimental.pallas import tpu as pltpu
from jax.experimental.pallas import tpu_sc as plsc

N = 4096
K = 1024
D = 32
TOPK = 8

NW = 32
RPW = N // NW
L = 16
SEG = K // L
RADIX = 32
NPASS = 7
RQ = 8

_ZVALS = tuple(float(np.float32(1.0) / np.float32(r + 1)) for r in range(TOPK))


def _dist_body(x_ref, ct_ref, d_ref):
    x = x_ref[...]
    ct = ct_ref[...]
    x2 = jnp.sum(x * x, axis=1, keepdims=True)
    c2 = jnp.sum(ct * ct, axis=0, keepdims=True)
    xc = lax.dot_general(x, ct, (((1,), (0,)), ((), ())),
                         precision=lax.Precision.HIGHEST,
                         preferred_element_type=jnp.float32)
    d_ref[...] = jnp.sqrt(jnp.maximum(x2 + c2 - 2.0 * xc, 0.0))


def _xc_body(k_ref, c_ref, xc_ref):
    oh = jnp.where(k_ref[...] == 0, 1.0, 0.0)
    xc_ref[...] = lax.dot_general(oh, c_ref[...], (((1,), (0,)), ((), ())),
                                  precision=lax.Precision.HIGHEST,
                                  preferred_element_type=jnp.float32)


def _sc_body(d_hbm, isort_hbm, k_hbm, z_hbm, *scratch):
    dbufs = scratch[0:RQ]
    payas = scratch[RQ:2 * RQ]
    paybs = scratch[2 * RQ:3 * RQ]
    offss = scratch[3 * RQ:4 * RQ]
    sincls = scratch[4 * RQ:5 * RQ]
    sdigs = scratch[5 * RQ:6 * RQ]
    kbufs = scratch[6 * RQ:7 * RQ]
    zbufs = scratch[7 * RQ:8 * RQ]

    wid = lax.axis_index("s") * 2 + lax.axis_index("c")
    row0 = wid * RPW
    lane = lax.iota(jnp.int32, L)
    seg_base = lane * SEG
    zero16 = jnp.zeros((L,), jnp.int32)
    one16 = jnp.ones((L,), jnp.int32)
    zvals = jnp.zeros((L,), jnp.float32)
    for r in range(TOPK):
        zvals = jnp.where(lane == r, _ZVALS[r], zvals)

    def quad_body(qi, _):
        row = row0 + qi * RQ
        for j in range(RQ):
            pltpu.sync_copy(d_hbm.at[row + j], dbufs[j])

        def one_pass(shift, srcs, dsts, first):
            def zero_step(b, _):
                for j in range(RQ):
                    offss[j][pl.ds(b * L, L)] = zero16
                return 0

            lax.fori_loop(0, RADIX, zero_step, 0)

            def hist_step(s, _):
                idx = seg_base + s
                for j in range(RQ):
                    if first:
                        pv = idx
                    else:
                        pv = plsc.load_gather(srcs[j], [idx])
                    kv = plsc.bitcast(
                        plsc.load_gather(dbufs[j], [pv]), jnp.int32)
                    digit = lax.shift_right_logical(kv, shift) & (RADIX - 1)
                    hidx = digit * L + lane
                    plsc.addupdate_scatter(offss[j], [hidx], one16)
                return 0

            lax.fori_loop(0, SEG, hist_step, 0)

            def scan_step(b, _):
                for j in range(RQ):
                    v = offss[j][pl.ds(b * L, L)]
                    cs = plsc.cumsum(v)
                    sincls[j][pl.ds(b * L, L)] = cs
                    offss[j][pl.ds(b * L, L)] = cs - v
                return 0

            lax.fori_loop(0, RADIX, scan_step, 0)
            for j in range(RQ):
                t0 = plsc.load_gather(sincls[j], [lane * L + (L - 1)])
                t1 = plsc.load_gather(
                    sincls[j], [RADIX * L // 2 + lane * L + (L - 1)])
                cs0 = plsc.cumsum(t0)
                carry0 = jnp.sum(t0)
                sdigs[j][pl.ds(0, L)] = cs0 - t0
                sdigs[j][pl.ds(L, L)] = plsc.cumsum(t1) - t1 + carry0

            def perm_step(s, _):
                idx = seg_base + s
                for j in range(RQ):
                    if first:
                        pv = idx
                    else:
                        pv = plsc.load_gather(srcs[j], [idx])
                    kv = plsc.bitcast(
                        plsc.load_gather(dbufs[j], [pv]), jnp.int32)
                    digit = lax.shift_right_logical(kv, shift) & (RADIX - 1)
                    hidx = digit * L + lane
                    o1 = plsc.load_gather(offss[j], [hidx])
                    plsc.store_scatter(offss[j], [hidx], o1 + 1)
                    pos = o1 + plsc.load_gather(sdigs[j], [digit])
                    plsc.store_scatter(dsts[j], [pos], pv)
                return 0

            lax.fori_loop(0, SEG, perm_step, 0)

        one_pass(0, None, payas, True)
        for p in range(1, NPASS):
            one_pass(5 * p, (payas, paybs)[(p + 1) % 2],
                     (payas, paybs)[p % 2], False)
        fin = payas

        def out_step(s, _):
            pos = s * L + lane
            zv = jnp.where(s == 0, zvals, jnp.zeros((L,), jnp.float32))
            for j in range(RQ):
                pv = fin[j][pl.ds(s * L, L)]
                plsc.store_scatter(kbufs[j], [pv], pos)
                plsc.store_scatter(zbufs[j], [pv], zv)
            return 0

        lax.fori_loop(0, SEG, out_step, 0)
        for j in range(RQ):
            pltpu.sync_copy(fin[j], isort_hbm.at[row + j])
            pltpu.sync_copy(kbufs[j], k_hbm.at[row + j])
            pltpu.sync_copy(zbufs[j], z_hbm.at[row + j])
        return 0

    lax.fori_loop(0, RPW // RQ, quad_body, 0)


@jax.jit
def kernel(x, c):
    ct = c.T
    d = pl.pallas_call(
        _dist_body,
        grid=(N // 256,),
        in_specs=[
            pl.BlockSpec((256, D), lambda i: (i, 0)),
            pl.BlockSpec((D, K), lambda i: (0, 0)),
        ],
        out_specs=pl.BlockSpec((256, K), lambda i: (i, 0)),
        out_shape=jax.ShapeDtypeStruct((N, K), jnp.float32),
    )(x, ct)

    scratch = (
        [pltpu.VMEM((K,), jnp.float32) for _ in range(RQ)]
        + [pltpu.VMEM((K,), jnp.int32) for _ in range(RQ)]
        + [pltpu.VMEM((K,), jnp.int32) for _ in range(RQ)]
        + [pltpu.VMEM((RADIX * L,), jnp.int32) for _ in range(RQ)]
        + [pltpu.VMEM((RADIX * L,), jnp.int32) for _ in range(RQ)]
        + [pltpu.VMEM((RADIX,), jnp.int32) for _ in range(RQ)]
        + [pltpu.VMEM((K,), jnp.int32) for _ in range(RQ)]
        + [pltpu.VMEM((K,), jnp.float32) for _ in range(RQ)]
    )
    sc_fn = functools.partial(
        pl.kernel,
        mesh=plsc.VectorSubcoreMesh(core_axis_name="c", subcore_axis_name="s"),
        out_type=[
            jax.ShapeDtypeStruct((N, K), jnp.int32),
            jax.ShapeDtypeStruct((N, K), jnp.int32),
            jax.ShapeDtypeStruct((N, K), jnp.float32),
        ],
        scratch_types=scratch,
        compiler_params=pltpu.CompilerParams(needs_layout_passes=False),
    )(_sc_body)
    isort, k, z = sc_fn(d)

    x_c = pl.pallas_call(
        _xc_body,
        grid=(N // 256,),
        in_specs=[
            pl.BlockSpec((256, K), lambda i: (i, 0)),
            pl.BlockSpec((K, D), lambda i: (0, 0)),
        ],
        out_specs=pl.BlockSpec((256, D), lambda i: (i, 0)),
        out_shape=jax.ShapeDtypeStruct((N, D), jnp.float32),
    )(k, c)
    return (d, isort, k, z, x_c)
